# baseline (speedup 1.0000x reference)
import functools

import jax
import jax.numpy as jnp
from jax import lax
from jax.experimental import pallas as pl
from jax.experimental.pallas import tpu as pltpu
from jax.experimental.pallas import tpu_sc as plsc

_N = 10000
_D = 256
_H = 256
_HALF = 128
_E = 160000
_NTILES = 16
_CH = 128
_EPT = _E // _NTILES
_NCH = (_EPT + _CH - 1) // _CH
_RPT = 624
_TAIL0 = 15 * _RPT
_TAIL = _N - _TAIL0
_N_PAD = _N + 16
_SCALE = 0.7071067811865476
_BR = 1000
_NB = _N // _BR


def _segsum_body(xa, xb, srcT, dstT, outa, outb, acc, srcb, dstb, gbuf, gsem):
    cid = lax.axis_index("c")
    sid = lax.axis_index("s")
    r0 = sid * _RPT

    pltpu.sync_copy(srcT.at[sid], srcb)
    pltpu.sync_copy(dstT.at[sid], dstb)

    @pl.when(cid == 0)
    def _():
        @pl.when(sid < 15)
        def _():
            pltpu.sync_copy(xa.at[pl.ds(r0, _RPT)], acc.at[pl.ds(r0, _RPT)])

        @pl.when(sid == 15)
        def _():
            pltpu.sync_copy(xa.at[pl.ds(_TAIL0, _TAIL)],
                            acc.at[pl.ds(_TAIL0, _TAIL)])

    @pl.when(cid == 1)
    def _():
        @pl.when(sid < 15)
        def _():
            pltpu.sync_copy(xb.at[pl.ds(r0, _RPT)], acc.at[pl.ds(r0, _RPT)])

        @pl.when(sid == 15)
        def _():
            pltpu.sync_copy(xb.at[pl.ds(_TAIL0, _TAIL)],
                            acc.at[pl.ds(_TAIL0, _TAIL)])

    plsc.subcore_barrier()

    def chunk(j, carry):
        @pl.when(cid == 0)
        def _():
            pltpu.async_copy(xa.at[srcb.at[j]], gbuf, gsem).wait()

        @pl.when(cid == 1)
        def _():
            pltpu.async_copy(xb.at[srcb.at[j]], gbuf, gsem).wait()

        pltpu.sync_copy(gbuf, acc.at[dstb.at[j]], add=True)
        return carry

    lax.fori_loop(0, _NCH, chunk, 0)
    plsc.subcore_barrier()

    @pl.when(cid == 0)
    def _():
        @pl.when(sid < 15)
        def _():
            pltpu.sync_copy(acc.at[pl.ds(r0, _RPT)], outa.at[pl.ds(r0, _RPT)])

        @pl.when(sid == 15)
        def _():
            pltpu.sync_copy(acc.at[pl.ds(_TAIL0, _TAIL)],
                            outa.at[pl.ds(_TAIL0, _TAIL)])

    @pl.when(cid == 1)
    def _():
        @pl.when(sid < 15)
        def _():
            pltpu.sync_copy(acc.at[pl.ds(r0, _RPT)], outb.at[pl.ds(r0, _RPT)])

        @pl.when(sid == 15)
        def _():
            pltpu.sync_copy(acc.at[pl.ds(_TAIL0, _TAIL)],
                            outb.at[pl.ds(_TAIL0, _TAIL)])


@functools.cache
def _make_segsum():
    return pl.kernel(
        _segsum_body,
        out_type=(jax.ShapeDtypeStruct((_N, _HALF), jnp.float32),
                  jax.ShapeDtypeStruct((_N, _HALF), jnp.float32)),
        mesh=plsc.VectorSubcoreMesh(core_axis_name="c", subcore_axis_name="s"),
        scratch_types=[
            pltpu.VMEM_SHARED((_N_PAD, _HALF), jnp.float32),
            pltpu.VMEM((_NCH, _CH), jnp.int32),
            pltpu.VMEM((_NCH, _CH), jnp.int32),
            pltpu.VMEM((_CH, _HALF), jnp.float32),
            pltpu.SemaphoreType.DMA,
        ],
    )


def _fc_body(inp, w, b, outa, outb):
    x = jnp.dot(inp[...], w[...], preferred_element_type=jnp.float32) + b[...]
    outa[...] = x[:, :_HALF]
    outb[...] = x[:, _HALF:]


def _mid_body(sa, sb, xa, xb, w, b, outa, outb):
    h = (jnp.dot(sa[...], w[0:_HALF, :], preferred_element_type=jnp.float32)
         + jnp.dot(sb[...], w[_HALF:_H, :], preferred_element_type=jnp.float32)
         + b[...])
    outa[...] = (xa[...] + h[:, :_HALF]) * _SCALE
    outb[...] = (xb[...] + h[:, _HALF:]) * _SCALE


def _final_body(sa, sb, xa, xb, w, b, y_ref, sum_ref, sq_ref):
    i = pl.program_id(0)
    h = (jnp.dot(sa[...], w[0:_HALF, :], preferred_element_type=jnp.float32)
         + jnp.dot(sb[...], w[_HALF:_H, :], preferred_element_type=jnp.float32)
         + b[...])
    xfull = jnp.concatenate([xa[...], xb[...]], axis=1)
    y = (xfull + h) * _SCALE
    y_ref[...] = y

    @pl.when(i == 0)
    def _():
        sum_ref[...] = jnp.zeros_like(sum_ref)
        sq_ref[...] = jnp.zeros_like(sq_ref)

    sum_ref[...] += jnp.sum(y, axis=0, keepdims=True)
    sq_ref[...] += jnp.sum(y * y, axis=0, keepdims=True)


def _bn_body(y, ssum, ssq, g, bt, out):
    mean = ssum[...] / _N
    var = ssq[...] / _N - mean * mean
    rstd = lax.rsqrt(var + 1e-5)
    sc = rstd * g[...]
    sh = bt[...] - mean * sc
    out[...] = y[...] * sc + sh


def _row_spec(width):
    return pl.BlockSpec((_BR, width), lambda i: (i, 0))


def _full_spec(shape):
    return pl.BlockSpec(shape, lambda i: (0,) * len(shape))


def _fc(inputs, w, b):
    return pl.pallas_call(
        _fc_body,
        grid=(_NB,),
        in_specs=[_row_spec(_D), _full_spec((_D, _H)), _full_spec((1, _H))],
        out_specs=(_row_spec(_HALF), _row_spec(_HALF)),
        out_shape=(jax.ShapeDtypeStruct((_N, _HALF), jnp.float32),) * 2,
    )(inputs, w, b)


def _mid(sa, sb, xa, xb, w, b):
    return pl.pallas_call(
        _mid_body,
        grid=(_NB,),
        in_specs=[_row_spec(_HALF)] * 4 + [_full_spec((_H, _H)), _full_spec((1, _H))],
        out_specs=(_row_spec(_HALF), _row_spec(_HALF)),
        out_shape=(jax.ShapeDtypeStruct((_N, _HALF), jnp.float32),) * 2,
    )(sa, sb, xa, xb, w, b)


def _final(sa, sb, xa, xb, w, b):
    return pl.pallas_call(
        _final_body,
        grid=(_NB,),
        in_specs=[_row_spec(_HALF)] * 4 + [_full_spec((_H, _H)), _full_spec((1, _H))],
        out_specs=(_row_spec(_H), _full_spec((1, _H)), _full_spec((1, _H))),
        out_shape=(jax.ShapeDtypeStruct((_N, _H), jnp.float32),
                   jax.ShapeDtypeStruct((1, _H), jnp.float32),
                   jax.ShapeDtypeStruct((1, _H), jnp.float32)),
        compiler_params=pltpu.CompilerParams(
            dimension_semantics=("arbitrary",)),
    )(sa, sb, xa, xb, w, b)


def _bn(y, ssum, ssq, g, bt):
    return pl.pallas_call(
        _bn_body,
        grid=(_NB,),
        in_specs=[_row_spec(_H), _full_spec((1, _H)), _full_spec((1, _H)),
                  _full_spec((1, _H)), _full_spec((1, _H))],
        out_specs=_row_spec(_H),
        out_shape=jax.ShapeDtypeStruct((_N, _H), jnp.float32),
    )(y, ssum, ssq, g, bt)


def kernel(clm_all, inputs, fc_W, fc_b, W1, b1, W2, b2, gamma, beta):
    src0 = clm_all[0]
    order = jnp.argsort(src0)
    src = src0[order]
    dst = clm_all[1][order]
    pad = _NCH * _CH
    srcT = jnp.zeros((_NTILES, pad), jnp.int32).at[:, :_EPT].set(
        src.reshape(_NTILES, _EPT)).reshape(_NTILES, _NCH, _CH)
    dstT = jnp.full((_NTILES, pad), _N, jnp.int32).at[:, :_EPT].set(
        dst.reshape(_NTILES, _EPT)).reshape(_NTILES, _NCH, _CH)

    b_fc = fc_b.reshape(1, _H)
    b_1 = b1.reshape(1, _H)
    b_2 = b2.reshape(1, _H)
    g2 = gamma.reshape(1, _H)
    bt2 = beta.reshape(1, _H)

    segsum = _make_segsum()
    xa, xb = _fc(inputs, fc_W, b_fc)
    sa, sb = segsum(xa, xb, srcT, dstT)
    x1a, x1b = _mid(sa, sb, xa, xb, W1, b_1)
    s2a, s2b = segsum(x1a, x1b, srcT, dstT)
    y, ssum, ssq = _final(s2a, s2b, x1a, x1b, W2, b_2)
    return _bn(y, ssum, ssq, g2, bt2)


# speedup vs baseline: 1.7111x; 1.7111x over previous
import functools

import jax
import jax.numpy as jnp
from jax import lax
from jax.experimental import pallas as pl
from jax.experimental.pallas import tpu as pltpu
from jax.experimental.pallas import tpu_sc as plsc

_N = 10000
_D = 256
_H = 256
_HALF = 128
_E = 160000
_NTILES = 16
_CH = 128
_EPT = _E // _NTILES
_NCH = (_EPT + _CH - 1) // _CH
_RPT = 624
_TAIL0 = 15 * _RPT
_TAIL = _N - _TAIL0
_N_PAD = _N + 16
_SCALE = 0.7071067811865476
_BR = 1000
_NB = _N // _BR


def _segsum_body(xa, xb, srcT, dstT, outa, outb, acc, srcb, dstb, gbuf, gsem):
    cid = lax.axis_index("c")
    sid = lax.axis_index("s")
    r0 = sid * _RPT

    pltpu.sync_copy(srcT.at[sid], srcb)
    pltpu.sync_copy(dstT.at[sid], dstb)

    @pl.when(cid == 0)
    def _():
        @pl.when(sid < 15)
        def _():
            pltpu.sync_copy(xa.at[pl.ds(r0, _RPT)], acc.at[pl.ds(r0, _RPT)])

        @pl.when(sid == 15)
        def _():
            pltpu.sync_copy(xa.at[pl.ds(_TAIL0, _TAIL)],
                            acc.at[pl.ds(_TAIL0, _TAIL)])

    @pl.when(cid == 1)
    def _():
        @pl.when(sid < 15)
        def _():
            pltpu.sync_copy(xb.at[pl.ds(r0, _RPT)], acc.at[pl.ds(r0, _RPT)])

        @pl.when(sid == 15)
        def _():
            pltpu.sync_copy(xb.at[pl.ds(_TAIL0, _TAIL)],
                            acc.at[pl.ds(_TAIL0, _TAIL)])

    plsc.subcore_barrier()

    def chunk(j, carry):
        @pl.when(cid == 0)
        def _():
            pltpu.async_copy(xa.at[srcb.at[j]], gbuf, gsem).wait()

        @pl.when(cid == 1)
        def _():
            pltpu.async_copy(xb.at[srcb.at[j]], gbuf, gsem).wait()

        pltpu.sync_copy(gbuf, acc.at[dstb.at[j]], add=True)
        return carry

    lax.fori_loop(0, _NCH, chunk, 0)
    plsc.subcore_barrier()

    @pl.when(cid == 0)
    def _():
        @pl.when(sid < 15)
        def _():
            pltpu.sync_copy(acc.at[pl.ds(r0, _RPT)], outa.at[pl.ds(r0, _RPT)])

        @pl.when(sid == 15)
        def _():
            pltpu.sync_copy(acc.at[pl.ds(_TAIL0, _TAIL)],
                            outa.at[pl.ds(_TAIL0, _TAIL)])

    @pl.when(cid == 1)
    def _():
        @pl.when(sid < 15)
        def _():
            pltpu.sync_copy(acc.at[pl.ds(r0, _RPT)], outb.at[pl.ds(r0, _RPT)])

        @pl.when(sid == 15)
        def _():
            pltpu.sync_copy(acc.at[pl.ds(_TAIL0, _TAIL)],
                            outb.at[pl.ds(_TAIL0, _TAIL)])


@functools.cache
def _make_segsum():
    return pl.kernel(
        _segsum_body,
        out_type=(jax.ShapeDtypeStruct((_N, _HALF), jnp.float32),
                  jax.ShapeDtypeStruct((_N, _HALF), jnp.float32)),
        mesh=plsc.VectorSubcoreMesh(core_axis_name="c", subcore_axis_name="s"),
        scratch_types=[
            pltpu.VMEM_SHARED((_N_PAD, _HALF), jnp.float32),
            pltpu.VMEM((_NCH, _CH), jnp.int32),
            pltpu.VMEM((_NCH, _CH), jnp.int32),
            pltpu.VMEM((_CH, _HALF), jnp.float32),
            pltpu.SemaphoreType.DMA,
        ],
    )


def _fc_body(inp, w, b, outa, outb):
    x = jnp.dot(inp[...], w[...], preferred_element_type=jnp.float32) + b[...]
    outa[...] = x[:, :_HALF]
    outb[...] = x[:, _HALF:]


def _mid_body(sa, sb, xa, xb, w, b, outa, outb):
    h = (jnp.dot(sa[...], w[0:_HALF, :], preferred_element_type=jnp.float32)
         + jnp.dot(sb[...], w[_HALF:_H, :], preferred_element_type=jnp.float32)
         + b[...])
    outa[...] = (xa[...] + h[:, :_HALF]) * _SCALE
    outb[...] = (xb[...] + h[:, _HALF:]) * _SCALE


def _final_bn_body(sa, sb, xa, xb, w, b, g, bt, out, y_scr, sum_scr, sq_scr):
    p = pl.program_id(0)
    i = pl.program_id(1)

    @pl.when(p == 0)
    def _():
        h = (jnp.dot(sa[...], w[0:_HALF, :], preferred_element_type=jnp.float32)
             + jnp.dot(sb[...], w[_HALF:_H, :],
                       preferred_element_type=jnp.float32)
             + b[...])
        xfull = jnp.concatenate([xa[...], xb[...]], axis=1)
        y = (xfull + h) * _SCALE
        y_scr[pl.ds(i * _BR, _BR), :] = y

        @pl.when(i == 0)
        def _():
            sum_scr[...] = jnp.zeros_like(sum_scr)
            sq_scr[...] = jnp.zeros_like(sq_scr)

        sum_scr[...] += jnp.sum(y, axis=0, keepdims=True)
        sq_scr[...] += jnp.sum(y * y, axis=0, keepdims=True)

    @pl.when(p == 1)
    def _():
        mean = sum_scr[...] / _N
        var = sq_scr[...] / _N - mean * mean
        rstd = lax.rsqrt(var + 1e-5)
        sc = rstd * g[...]
        sh = bt[...] - mean * sc
        out[...] = y_scr[pl.ds(i * _BR, _BR), :] * sc + sh


def _row_spec(width):
    return pl.BlockSpec((_BR, width), lambda i: (i, 0))


def _full_spec(shape):
    return pl.BlockSpec(shape, lambda i: (0,) * len(shape))


def _fc(inputs, w, b):
    return pl.pallas_call(
        _fc_body,
        grid=(_NB,),
        in_specs=[_row_spec(_D), _full_spec((_D, _H)), _full_spec((1, _H))],
        out_specs=(_row_spec(_HALF), _row_spec(_HALF)),
        out_shape=(jax.ShapeDtypeStruct((_N, _HALF), jnp.float32),) * 2,
    )(inputs, w, b)


def _mid(sa, sb, xa, xb, w, b):
    return pl.pallas_call(
        _mid_body,
        grid=(_NB,),
        in_specs=[_row_spec(_HALF)] * 4 + [_full_spec((_H, _H)), _full_spec((1, _H))],
        out_specs=(_row_spec(_HALF), _row_spec(_HALF)),
        out_shape=(jax.ShapeDtypeStruct((_N, _HALF), jnp.float32),) * 2,
    )(sa, sb, xa, xb, w, b)


def _final_bn(sa, sb, xa, xb, w, b, g, bt):
    in_spec = pl.BlockSpec((_BR, _HALF), lambda p, i: (i * (1 - p), 0))
    wide = lambda shape: pl.BlockSpec(shape, lambda p, i: (0,) * len(shape))
    return pl.pallas_call(
        _final_bn_body,
        grid=(2, _NB),
        in_specs=[in_spec] * 4 + [wide((_H, _H)), wide((1, _H)),
                                  wide((1, _H)), wide((1, _H))],
        out_specs=pl.BlockSpec((_BR, _H), lambda p, i: (i * p, 0)),
        out_shape=jax.ShapeDtypeStruct((_N, _H), jnp.float32),
        scratch_shapes=[pltpu.VMEM((_N, _H), jnp.float32),
                        pltpu.VMEM((1, _H), jnp.float32),
                        pltpu.VMEM((1, _H), jnp.float32)],
        compiler_params=pltpu.CompilerParams(
            dimension_semantics=("arbitrary", "arbitrary")),
    )(sa, sb, xa, xb, w, b, g, bt)


def kernel(clm_all, inputs, fc_W, fc_b, W1, b1, W2, b2, gamma, beta):
    src = clm_all[0]
    dst = clm_all[1]
    pad = _NCH * _CH
    srcT = jnp.zeros((_NTILES, pad), jnp.int32).at[:, :_EPT].set(
        src.reshape(_NTILES, _EPT)).reshape(_NTILES, _NCH, _CH)
    dstT = jnp.full((_NTILES, pad), _N, jnp.int32).at[:, :_EPT].set(
        dst.reshape(_NTILES, _EPT)).reshape(_NTILES, _NCH, _CH)

    b_fc = fc_b.reshape(1, _H)
    b_1 = b1.reshape(1, _H)
    b_2 = b2.reshape(1, _H)
    g2 = gamma.reshape(1, _H)
    bt2 = beta.reshape(1, _H)

    segsum = _make_segsum()
    xa, xb = _fc(inputs, fc_W, b_fc)
    sa, sb = segsum(xa, xb, srcT, dstT)
    x1a, x1b = _mid(sa, sb, xa, xb, W1, b_1)
    s2a, s2b = segsum(x1a, x1b, srcT, dstT)
    return _final_bn(s2a, s2b, x1a, x1b, W2, b_2, g2, bt2)
